# baseline (speedup 1.0000x reference)
import functools

import jax
import jax.numpy as jnp
from jax import lax
from jax.experimental import pallas as pl
from jax.experimental.pallas import tpu as pltpu
from jax.experimental.pallas import tpu_sc as plsc

V_TILE = 2048


def _gather(table, idx):
    n, per_tok = idx.shape[0], table.shape[1]
    info = plsc.get_sparse_core_info()
    nw = info.num_cores * info.num_subcores
    b_per_w = n // nw
    mesh = plsc.VectorSubcoreMesh(core_axis_name="c", subcore_axis_name="s")

    chunk = 16

    @functools.partial(
        pl.kernel,
        out_type=jax.ShapeDtypeStruct((n, per_tok), jnp.float32),
        mesh=mesh,
        scratch_types=[
            pltpu.VMEM((b_per_w,), jnp.int32),
            pltpu.VMEM((b_per_w, per_tok), jnp.float32),
            pltpu.SemaphoreType.DMA,
        ],
    )
    def gather_kernel(idx_hbm, table_hbm, out_hbm, idx_s, rows_v, sem):
        wid = lax.axis_index("s") * info.num_cores + lax.axis_index("c")
        base = wid * b_per_w
        pltpu.sync_copy(idx_hbm.at[pl.ds(base, b_per_w)], idx_s)

        def body(ci, _):
            base_i = ci * chunk
            v = idx_s[pl.ds(base_i, chunk)]
            copies = []
            for j in range(chunk):
                copies.append(
                    pltpu.async_copy(
                        table_hbm.at[v[j]], rows_v.at[base_i + j], sem))
            for c in copies:
                c.wait()
            return ()

        lax.fori_loop(0, b_per_w // chunk, body, (), unroll=False)
        pltpu.sync_copy(rows_v, out_hbm.at[pl.ds(base, b_per_w)])

    return gather_kernel(idx, table)


def _h1_body(emb_ref, w1_ref, b1_ref, h1_ref):
    h1 = lax.dot_general(
        emb_ref[...], w1_ref[...], (((1,), (1,)), ((), ())),
        preferred_element_type=jnp.float32)
    h1_ref[...] = jnp.maximum(h1 + b1_ref[...], 0.0).astype(jnp.bfloat16)


def _h1(emb, W1, b1):
    batch = emb.shape[0]
    hid = W1.shape[0]
    return pl.pallas_call(
        _h1_body,
        out_shape=jax.ShapeDtypeStruct((batch, hid), jnp.bfloat16),
    )(emb, W1, b1.reshape(1, hid))


NQ = 4


def _mm2_body(vocab, batch, h1_ref, w2_ref, b2_ref, out_ref,
              obuf, tbuf, sems, tsems):
    i = pl.program_id(0)
    n = pl.num_programs(0)
    slot = lax.rem(i, 2)
    rows = batch // NQ
    tail = vocab - (n - 1) * V_TILE

    def waitq(s, width):
        for q in range(NQ):
            pltpu.make_async_copy(
                obuf.at[s, pl.ds(q * rows, rows), pl.ds(0, width)],
                out_ref.at[pl.ds(q * rows, rows), pl.ds(0, width)],
                sems.at[s, q]).wait()

    @pl.when(i >= 2)
    def _():
        waitq(slot, V_TILE)

    w2b = w2_ref[...].astype(jnp.bfloat16)
    res = lax.dot_general(
        h1_ref[...], w2b, (((1,), (1,)), ((), ())),
        preferred_element_type=jnp.float32) + b2_ref[...]

    @pl.when(i < n - 1)
    def _():
        obuf[slot] = res
        off = i * V_TILE
        for q in range(NQ):
            pltpu.make_async_copy(
                obuf.at[slot, pl.ds(q * rows, rows)],
                out_ref.at[pl.ds(q * rows, rows), pl.ds(off, V_TILE)],
                sems.at[slot, q]).start()

    @pl.when(i == n - 1)
    def _():
        tbuf[...] = res[:, :tail]
        off = (n - 1) * V_TILE
        for q in range(NQ):
            pltpu.make_async_copy(
                tbuf.at[pl.ds(q * rows, rows)],
                out_ref.at[pl.ds(q * rows, rows), pl.ds(off, tail)],
                tsems.at[q]).start()
        waitq(1 - slot, V_TILE)
        for q in range(NQ):
            pltpu.make_async_copy(
                tbuf.at[pl.ds(q * rows, rows)],
                out_ref.at[pl.ds(q * rows, rows), pl.ds(off, tail)],
                tsems.at[q]).wait()


def _mm2(h1b, W2, b2):
    batch, hid = h1b.shape
    vocab = W2.shape[0]
    n = pl.cdiv(vocab, V_TILE)
    tail = vocab - (n - 1) * V_TILE
    body = functools.partial(_mm2_body, vocab, batch)
    return pl.pallas_call(
        body,
        grid=(n,),
        in_specs=[
            pl.BlockSpec((batch, hid), lambda i: (0, 0)),
            pl.BlockSpec((V_TILE, hid), lambda i: (i, 0)),
            pl.BlockSpec((1, V_TILE), lambda i: (0, i)),
        ],
        out_specs=pl.BlockSpec(memory_space=pl.ANY),
        out_shape=jax.ShapeDtypeStruct((batch, vocab), jnp.float32),
        scratch_shapes=[
            pltpu.VMEM((2, batch, V_TILE), jnp.float32),
            pltpu.VMEM((batch, tail), jnp.float32),
            pltpu.SemaphoreType.DMA((2, NQ)),
            pltpu.SemaphoreType.DMA((NQ,)),
        ],
    )(h1b, W2, b2.reshape(1, vocab))


def kernel(inputs, table, W1, b1, W2, b2):
    batch, ctx = inputs.shape
    idx = inputs.reshape(-1).astype(jnp.int32)
    emb = _gather(table, idx).reshape(batch, ctx * table.shape[1])
    h1b = _h1(emb, W1, b1)
    return _mm2(h1b, W2, b2)


# speedup vs baseline: 1.0015x; 1.0015x over previous
import functools

import jax
import jax.numpy as jnp
from jax import lax
from jax.experimental import pallas as pl
from jax.experimental.pallas import tpu as pltpu
from jax.experimental.pallas import tpu_sc as plsc

V_TILE = 2048


def _gather(table, idx):
    n, per_tok = idx.shape[0], table.shape[1]
    info = plsc.get_sparse_core_info()
    nw = info.num_cores * info.num_subcores
    b_per_w = n // nw
    mesh = plsc.VectorSubcoreMesh(core_axis_name="c", subcore_axis_name="s")

    chunk = 16

    @functools.partial(
        pl.kernel,
        out_type=jax.ShapeDtypeStruct((n, per_tok), jnp.float32),
        mesh=mesh,
        scratch_types=[
            pltpu.VMEM((b_per_w,), jnp.int32),
            pltpu.VMEM((b_per_w, per_tok), jnp.float32),
            pltpu.SemaphoreType.DMA,
        ],
    )
    def gather_kernel(idx_hbm, table_hbm, out_hbm, idx_s, rows_v, sem):
        wid = lax.axis_index("s") * info.num_cores + lax.axis_index("c")
        base = wid * b_per_w
        pltpu.sync_copy(idx_hbm.at[pl.ds(base, b_per_w)], idx_s)

        def body(ci, _):
            base_i = ci * chunk
            v = idx_s[pl.ds(base_i, chunk)]
            copies = []
            for j in range(chunk):
                copies.append(
                    pltpu.async_copy(
                        table_hbm.at[v[j]], rows_v.at[base_i + j], sem))
            for c in copies:
                c.wait()
            return ()

        lax.fori_loop(0, b_per_w // chunk, body, (), unroll=False)
        pltpu.sync_copy(rows_v, out_hbm.at[pl.ds(base, b_per_w)])

    return gather_kernel(idx, table)


def _h1_body(emb_ref, w1_ref, b1_ref, h1_ref):
    h1 = lax.dot_general(
        emb_ref[...], w1_ref[...], (((1,), (1,)), ((), ())),
        preferred_element_type=jnp.float32)
    h1_ref[...] = jnp.maximum(h1 + b1_ref[...], 0.0).astype(jnp.bfloat16)


def _h1(emb, W1, b1):
    batch = emb.shape[0]
    hid = W1.shape[0]
    return pl.pallas_call(
        _h1_body,
        out_shape=jax.ShapeDtypeStruct((batch, hid), jnp.bfloat16),
    )(emb, W1, b1.reshape(1, hid))


NQ = 4
NSLOT = 4


def _mm2_body(vocab, batch, n_static, h1_ref, w2_ref, b2_ref, out_ref,
              obuf, tbuf, sems, tsems):
    i = pl.program_id(0)
    n = pl.num_programs(0)
    slot = lax.rem(i, NSLOT)
    rows = batch // NQ
    tail = vocab - (n - 1) * V_TILE

    def waitq(s, width):
        for q in range(NQ):
            pltpu.make_async_copy(
                obuf.at[s, pl.ds(q * rows, rows), pl.ds(0, width)],
                out_ref.at[pl.ds(q * rows, rows), pl.ds(0, width)],
                sems.at[s, q]).wait()

    @pl.when(i >= NSLOT)
    def _():
        waitq(slot, V_TILE)

    w2b = w2_ref[...].astype(jnp.bfloat16)
    res = lax.dot_general(
        h1_ref[...], w2b, (((1,), (1,)), ((), ())),
        preferred_element_type=jnp.float32) + b2_ref[...]

    @pl.when(i < n - 1)
    def _():
        obuf[slot] = res
        off = i * V_TILE
        for q in range(NQ):
            pltpu.make_async_copy(
                obuf.at[slot, pl.ds(q * rows, rows)],
                out_ref.at[pl.ds(q * rows, rows), pl.ds(off, V_TILE)],
                sems.at[slot, q]).start()

    @pl.when(i == n - 1)
    def _():
        tbuf[...] = res[:, :tail]
        off = (n - 1) * V_TILE
        for q in range(NQ):
            pltpu.make_async_copy(
                tbuf.at[pl.ds(q * rows, rows)],
                out_ref.at[pl.ds(q * rows, rows), pl.ds(off, tail)],
                tsems.at[q]).start()
        last_slot = (n_static - 1) % NSLOT
        for s in range(NSLOT):
            if s != last_slot:
                waitq(s, V_TILE)
        for q in range(NQ):
            pltpu.make_async_copy(
                tbuf.at[pl.ds(q * rows, rows)],
                out_ref.at[pl.ds(q * rows, rows), pl.ds(off, tail)],
                tsems.at[q]).wait()


def _mm2(h1b, W2, b2):
    batch, hid = h1b.shape
    vocab = W2.shape[0]
    n = pl.cdiv(vocab, V_TILE)
    tail = vocab - (n - 1) * V_TILE
    body = functools.partial(_mm2_body, vocab, batch, n)
    return pl.pallas_call(
        body,
        grid=(n,),
        in_specs=[
            pl.BlockSpec((batch, hid), lambda i: (0, 0)),
            pl.BlockSpec((V_TILE, hid), lambda i: (i, 0)),
            pl.BlockSpec((1, V_TILE), lambda i: (0, i)),
        ],
        out_specs=pl.BlockSpec(memory_space=pl.ANY),
        out_shape=jax.ShapeDtypeStruct((batch, vocab), jnp.float32),
        scratch_shapes=[
            pltpu.VMEM((NSLOT, batch, V_TILE), jnp.float32),
            pltpu.VMEM((batch, tail), jnp.float32),
            pltpu.SemaphoreType.DMA((NSLOT, NQ)),
            pltpu.SemaphoreType.DMA((NQ,)),
        ],
    )(h1b, W2, b2.reshape(1, vocab))


def kernel(inputs, table, W1, b1, W2, b2):
    batch, ctx = inputs.shape
    idx = inputs.reshape(-1).astype(jnp.int32)
    emb = _gather(table, idx).reshape(batch, ctx * table.shape[1])
    h1b = _h1(emb, W1, b1)
    return _mm2(h1b, W2, b2)
